# baseline (speedup 1.0000x reference)
import jax
import jax.numpy as jnp
from jax.experimental import pallas as pl
from jax.experimental.pallas import tpu as pltpu

_B = 2048
_CS = 15
_QS = 10
_PE = 50
_PN = 2 * _CS
_D = 128
_G = 8
_TEMP = 0.1
_ITERS = 20
_NPROP = 3


def _body(fi_ref, ti_ref, tiT_ref, mk_ref, nf_ref, ef_ref,
          wen_ref, ben_ref, wee_ref, bee_ref,
          w12_ref, w3_ref, bm_ref,
          u1_ref, u2_ref, bu_ref,
          wf1_ref, bf1_ref, wf2_ref, bf2_ref,
          out_ref):
    i = pl.program_id(0)
    NB = _G * _PN
    EB = _G * _PE

    fi = fi_ref[0] - i * NB
    ti = ti_ref[0] - i * NB
    tiT = tiT_ref[0] - i * NB
    mk = mk_ref[0]

    cols = jax.lax.broadcasted_iota(jnp.int32, (EB, NB), 1)
    oh_f = (cols == fi).astype(jnp.bfloat16)
    oh_t = (cols == ti).astype(jnp.bfloat16)
    rows = jax.lax.broadcasted_iota(jnp.int32, (NB, EB), 0)
    oh_tT = (rows == tiT).astype(jnp.bfloat16)

    bf = jnp.bfloat16
    dotb = lambda x, y: jnp.dot(x.astype(bf), y,
                                preferred_element_type=jnp.float32)

    def dot01(oh, x):
        hi = x.astype(bf)
        r1 = x - hi.astype(jnp.float32)
        mid = r1.astype(bf)
        lo = (r1 - mid.astype(jnp.float32)).astype(bf)
        return (jnp.dot(oh, hi, preferred_element_type=jnp.float32)
                + jnp.dot(oh, mid, preferred_element_type=jnp.float32)
                + jnp.dot(oh, lo, preferred_element_type=jnp.float32))

    h = nf_ref[0] * wen_ref[...] + ben_ref[...]
    e_enc = ef_ref[0] * wee_ref[...] + bee_ref[...]
    w12 = w12_ref[...].astype(bf)
    u1 = u1_ref[...].astype(bf)
    u2 = u2_ref[...].astype(bf)
    ce = dotb(e_enc, w3_ref[...].astype(bf)) + bm_ref[...]

    for _ in range(_NPROP):
        hb = h.astype(bf)
        hf = jnp.dot(oh_f, hb, preferred_element_type=jnp.float32)
        ht = jnp.dot(oh_t, hb, preferred_element_type=jnp.float32)
        hft = jnp.concatenate([hf, ht], axis=1)
        msg = jnp.maximum(
            jnp.dot(hft.astype(bf), w12,
                    preferred_element_type=jnp.float32) + ce, 0.0) * mk
        agg = dot01(oh_tT, msg)
        h = jnp.maximum(
            dotb(agg, u1)
            + jnp.dot(hb, u2, preferred_element_type=jnp.float32)
            + bu_ref[...], 0.0)

    t1 = jnp.maximum(dotb(h, wf1_ref[...].astype(bf)) + bf1_ref[...], 0.0)
    t2 = dotb(t1, wf2_ref[...].astype(bf)) + bf2_ref[...]

    r = jax.lax.broadcasted_iota(jnp.int32, (NB, 1), 0) % _PN
    mq = t2 * (r < _QS).astype(jnp.float32)
    mc = t2 * (r >= _CS).astype(jnp.float32)
    s_big = jax.lax.dot_general(mq.astype(bf), mc.astype(bf),
                                (((1,), (1,)), ((), ())),
                                preferred_element_type=jnp.float32)

    blocks = [
        s_big[g * _PN:g * _PN + _CS, g * _PN + _CS:(g + 1) * _PN][None]
        for g in range(_G)
    ]
    out_ref[...] = jnp.concatenate(blocks, axis=0)


_G2 = 128


def _sink_body(s_ref, out_ref):
    sin = s_ref[...]
    la = jnp.stack([jnp.transpose(sin[:, q, :]) for q in range(_CS)],
                   axis=0) / _TEMP

    for _ in range(_ITERS):
        m = jnp.max(la, axis=1, keepdims=True)
        la = la - (jnp.log(jnp.sum(jnp.exp(la - m), axis=1, keepdims=True)) + m)
        m = jnp.max(la, axis=0, keepdims=True)
        la = la - (jnp.log(jnp.sum(jnp.exp(la - m), axis=0, keepdims=True)) + m)

    p = jnp.exp(la)
    for q in range(_CS):
        out_ref[:, q, :] = jnp.transpose(p[q])


def kernel(node_features, edge_features, mask_from_idx,
           W_enc_n, b_enc_n, W_enc_e, b_enc_e,
           W_msg, b_msg, W_upd, b_upd,
           W_fc1, b_fc1, W_fc2, b_fc2,
           from_idx, to_idx, graph_idx):
    del graph_idx
    S = _B // _G
    EB = _G * _PE
    NB = _G * _PN
    E_real = _B * _PE

    fi = from_idx[:E_real].astype(jnp.int32).reshape(S, EB, 1)
    ti3 = to_idx[:E_real].astype(jnp.int32).reshape(S, EB, 1)
    tiT = to_idx[:E_real].astype(jnp.int32).reshape(S, 1, EB)
    mk = mask_from_idx[:E_real].reshape(S, EB, 1)
    nf = node_features.reshape(S, NB, 1)
    ef = edge_features[:E_real].reshape(S, EB, 1)

    W12, W3 = W_msg[:2 * _D], W_msg[2 * _D:]
    U1, U2 = W_upd[:_D], W_upd[_D:]
    row = lambda v: v.reshape(1, -1)

    def step_spec(rows_, cols_):
        return pl.BlockSpec((1, rows_, cols_), lambda i: (i, 0, 0))

    def full_spec(shape):
        return pl.BlockSpec(shape, lambda i: tuple(0 for _ in shape))

    operands = [fi, ti3, tiT, mk, nf, ef,
                W_enc_n, row(b_enc_n), W_enc_e, row(b_enc_e),
                W12, W3, row(b_msg),
                U1, U2, row(b_upd),
                W_fc1, row(b_fc1), W_fc2, row(b_fc2)]
    specs = [step_spec(EB, 1), step_spec(EB, 1), step_spec(1, EB),
             step_spec(EB, 1), step_spec(NB, 1), step_spec(EB, 1)]
    specs += [full_spec(o.shape) for o in operands[6:]]

    s = pl.pallas_call(
        _body,
        grid=(S,),
        in_specs=specs,
        out_specs=pl.BlockSpec((_G, _CS, _CS), lambda i: (i, 0, 0)),
        out_shape=jax.ShapeDtypeStruct((_B, _CS, _CS), jnp.float32),
        compiler_params=pltpu.CompilerParams(
            dimension_semantics=("parallel",)),
    )(*operands)

    return pl.pallas_call(
        _sink_body,
        grid=(_B // _G2,),
        in_specs=[pl.BlockSpec((_G2, _CS, _CS), lambda i: (i, 0, 0))],
        out_specs=pl.BlockSpec((_G2, _CS, _CS), lambda i: (i, 0, 0)),
        out_shape=jax.ShapeDtypeStruct((_B, _CS, _CS), jnp.float32),
        compiler_params=pltpu.CompilerParams(
            dimension_semantics=("parallel",)),
    )(s)


# speedup vs baseline: 1.0544x; 1.0544x over previous
import jax
import jax.numpy as jnp
from jax.experimental import pallas as pl
from jax.experimental.pallas import tpu as pltpu

_B = 2048
_CS = 15
_QS = 10
_PE = 50
_PN = 2 * _CS
_D = 128
_G = 8
_H = 2
_TEMP = 0.1
_ITERS = 20
_NPROP = 3


def _group(fi, ti, tiT, mk, nf, ef, base,
           wen, ben, wee, bee, w12, w3, bm, u1, u2, bu,
           wf1, bf1, wf2, bf2):
    NB = _G * _PN
    EB = _G * _PE
    bf = jnp.bfloat16
    dotb = lambda x, y: jnp.dot(x.astype(bf), y,
                                preferred_element_type=jnp.float32)

    def dot01(oh, x):
        hi = x.astype(bf)
        r1 = x - hi.astype(jnp.float32)
        mid = r1.astype(bf)
        lo = (r1 - mid.astype(jnp.float32)).astype(bf)
        return (jnp.dot(oh, hi, preferred_element_type=jnp.float32)
                + jnp.dot(oh, mid, preferred_element_type=jnp.float32)
                + jnp.dot(oh, lo, preferred_element_type=jnp.float32))

    fi = fi - base
    ti = ti - base
    tiT = tiT - base
    cols = jax.lax.broadcasted_iota(jnp.int32, (EB, NB), 1)
    oh_f = (cols == fi).astype(bf)
    oh_t = (cols == ti).astype(bf)
    rows = jax.lax.broadcasted_iota(jnp.int32, (NB, EB), 0)
    oh_tT = (rows == tiT).astype(bf)

    h = nf * wen + ben
    e_enc = ef * wee + bee
    ce = dotb(e_enc, w3) + bm

    for _ in range(_NPROP):
        hb = h.astype(bf)
        hf = jnp.dot(oh_f, hb, preferred_element_type=jnp.float32)
        ht = jnp.dot(oh_t, hb, preferred_element_type=jnp.float32)
        hft = jnp.concatenate([hf, ht], axis=1)
        msg = jnp.maximum(
            jnp.dot(hft.astype(bf), w12,
                    preferred_element_type=jnp.float32) + ce, 0.0) * mk
        agg = dot01(oh_tT, msg)
        h = jnp.maximum(
            dotb(agg, u1)
            + jnp.dot(hb, u2, preferred_element_type=jnp.float32)
            + bu, 0.0)

    t1 = jnp.maximum(dotb(h, wf1) + bf1, 0.0)
    t2 = dotb(t1, wf2) + bf2

    r = jax.lax.broadcasted_iota(jnp.int32, (NB, 1), 0) % _PN
    mq = t2 * (r < _QS).astype(jnp.float32)
    mc = t2 * (r >= _CS).astype(jnp.float32)
    s_big = jax.lax.dot_general(mq.astype(bf), mc.astype(bf),
                                (((1,), (1,)), ((), ())),
                                preferred_element_type=jnp.float32)

    return [
        s_big[g * _PN:g * _PN + _CS, g * _PN + _CS:(g + 1) * _PN][None]
        for g in range(_G)
    ]


def _body(fi_ref, ti_ref, tiT_ref, mk_ref, nf_ref, ef_ref,
          wen_ref, ben_ref, wee_ref, bee_ref,
          w12_ref, w3_ref, bm_ref,
          u1_ref, u2_ref, bu_ref,
          wf1_ref, bf1_ref, wf2_ref, bf2_ref,
          out_ref):
    i = pl.program_id(0)
    NB = _G * _PN
    EB = _G * _PE
    bf = jnp.bfloat16
    w12 = w12_ref[...].astype(bf)
    w3 = w3_ref[...].astype(bf)
    u1 = u1_ref[...].astype(bf)
    u2 = u2_ref[...].astype(bf)
    wf1 = wf1_ref[...].astype(bf)
    wf2 = wf2_ref[...].astype(bf)
    blocks = []
    for half in range(_H):
        es = slice(half * EB, (half + 1) * EB)
        ns = slice(half * NB, (half + 1) * NB)
        blocks += _group(
            fi_ref[0][es], ti_ref[0][es], tiT_ref[0][:, es], mk_ref[0][es],
            nf_ref[0][ns], ef_ref[0][es], i * _H * NB + half * NB,
            wen_ref[...], ben_ref[...], wee_ref[...], bee_ref[...],
            w12, w3, bm_ref[...], u1, u2, bu_ref[...],
            wf1, bf1_ref[...], wf2, bf2_ref[...])
    out_ref[...] = jnp.concatenate(blocks, axis=0)


_G2 = 128


def _sink_body(s_ref, out_ref):
    sin = s_ref[...]
    la = jnp.stack([jnp.transpose(sin[:, q, :]) for q in range(_CS)],
                   axis=0) / _TEMP

    for _ in range(_ITERS):
        m = jnp.max(la, axis=1, keepdims=True)
        la = la - (jnp.log(jnp.sum(jnp.exp(la - m), axis=1, keepdims=True)) + m)
        m = jnp.max(la, axis=0, keepdims=True)
        la = la - (jnp.log(jnp.sum(jnp.exp(la - m), axis=0, keepdims=True)) + m)

    p = jnp.exp(la)
    for q in range(_CS):
        out_ref[:, q, :] = jnp.transpose(p[q])


def kernel(node_features, edge_features, mask_from_idx,
           W_enc_n, b_enc_n, W_enc_e, b_enc_e,
           W_msg, b_msg, W_upd, b_upd,
           W_fc1, b_fc1, W_fc2, b_fc2,
           from_idx, to_idx, graph_idx):
    del graph_idx
    S = _B // (_G * _H)
    EB = _H * _G * _PE
    NB = _H * _G * _PN
    E_real = _B * _PE

    fi = from_idx[:E_real].astype(jnp.int32).reshape(S, EB, 1)
    ti3 = to_idx[:E_real].astype(jnp.int32).reshape(S, EB, 1)
    tiT = to_idx[:E_real].astype(jnp.int32).reshape(S, 1, EB)
    mk = mask_from_idx[:E_real].reshape(S, EB, 1)
    nf = node_features.reshape(S, NB, 1)
    ef = edge_features[:E_real].reshape(S, EB, 1)

    W12, W3 = W_msg[:2 * _D], W_msg[2 * _D:]
    U1, U2 = W_upd[:_D], W_upd[_D:]
    row = lambda v: v.reshape(1, -1)

    def step_spec(rows_, cols_):
        return pl.BlockSpec((1, rows_, cols_), lambda i: (i, 0, 0))

    def full_spec(shape):
        return pl.BlockSpec(shape, lambda i: tuple(0 for _ in shape))

    operands = [fi, ti3, tiT, mk, nf, ef,
                W_enc_n, row(b_enc_n), W_enc_e, row(b_enc_e),
                W12, W3, row(b_msg),
                U1, U2, row(b_upd),
                W_fc1, row(b_fc1), W_fc2, row(b_fc2)]
    specs = [step_spec(EB, 1), step_spec(EB, 1), step_spec(1, EB),
             step_spec(EB, 1), step_spec(NB, 1), step_spec(EB, 1)]
    specs += [full_spec(o.shape) for o in operands[6:]]

    s = pl.pallas_call(
        _body,
        grid=(S,),
        in_specs=specs,
        out_specs=pl.BlockSpec((_H * _G, _CS, _CS), lambda i: (i, 0, 0)),
        out_shape=jax.ShapeDtypeStruct((_B, _CS, _CS), jnp.float32),
        compiler_params=pltpu.CompilerParams(
            dimension_semantics=("parallel",)),
    )(*operands)

    return pl.pallas_call(
        _sink_body,
        grid=(_B // _G2,),
        in_specs=[pl.BlockSpec((_G2, _CS, _CS), lambda i: (i, 0, 0))],
        out_specs=pl.BlockSpec((_G2, _CS, _CS), lambda i: (i, 0, 0)),
        out_shape=jax.ShapeDtypeStruct((_B, _CS, _CS), jnp.float32),
        compiler_params=pltpu.CompilerParams(
            dimension_semantics=("parallel",)),
    )(s)


# speedup vs baseline: 1.0772x; 1.0216x over previous
import jax
import jax.numpy as jnp
from jax.experimental import pallas as pl
from jax.experimental.pallas import tpu as pltpu

_B = 2048
_CS = 15
_QS = 10
_PE = 50
_PN = 2 * _CS
_D = 128
_G = 8
_H = 4
_TEMP = 0.1
_ITERS = 20
_NPROP = 3


def _group(fi, ti, tiT, mk, nf, ef, base,
           wen, ben, wee, bee, w12, w3, bm, u1, u2, bu,
           wf1, bf1, wf2, bf2):
    NB = _G * _PN
    EB = _G * _PE
    bf = jnp.bfloat16
    dotb = lambda x, y: jnp.dot(x.astype(bf), y,
                                preferred_element_type=jnp.float32)

    def dot01(oh, x):
        hi = x.astype(bf)
        r1 = x - hi.astype(jnp.float32)
        mid = r1.astype(bf)
        lo = (r1 - mid.astype(jnp.float32)).astype(bf)
        return (jnp.dot(oh, hi, preferred_element_type=jnp.float32)
                + jnp.dot(oh, mid, preferred_element_type=jnp.float32)
                + jnp.dot(oh, lo, preferred_element_type=jnp.float32))

    fi = fi - base
    ti = ti - base
    tiT = tiT - base
    cols = jax.lax.broadcasted_iota(jnp.int32, (EB, NB), 1)
    oh_f = (cols == fi).astype(bf)
    oh_t = (cols == ti).astype(bf)
    rows = jax.lax.broadcasted_iota(jnp.int32, (NB, EB), 0)
    oh_tT = (rows == tiT).astype(bf)

    h = nf * wen + ben
    e_enc = ef * wee + bee
    ce = dotb(e_enc, w3) + bm

    for _ in range(_NPROP):
        hb = h.astype(bf)
        hf = jnp.dot(oh_f, hb, preferred_element_type=jnp.float32)
        ht = jnp.dot(oh_t, hb, preferred_element_type=jnp.float32)
        hft = jnp.concatenate([hf, ht], axis=1)
        msg = jnp.maximum(
            jnp.dot(hft.astype(bf), w12,
                    preferred_element_type=jnp.float32) + ce, 0.0) * mk
        agg = dot01(oh_tT, msg)
        h = jnp.maximum(
            dotb(agg, u1)
            + jnp.dot(hb, u2, preferred_element_type=jnp.float32)
            + bu, 0.0)

    t1 = jnp.maximum(dotb(h, wf1) + bf1, 0.0)
    t2 = dotb(t1, wf2) + bf2

    r = jax.lax.broadcasted_iota(jnp.int32, (NB, 1), 0) % _PN
    mq = t2 * (r < _QS).astype(jnp.float32)
    mc = t2 * (r >= _CS).astype(jnp.float32)
    s_big = jax.lax.dot_general(mq.astype(bf), mc.astype(bf),
                                (((1,), (1,)), ((), ())),
                                preferred_element_type=jnp.float32)

    return [
        s_big[g * _PN:g * _PN + _CS, g * _PN + _CS:(g + 1) * _PN][None]
        for g in range(_G)
    ]


def _body(fi_ref, ti_ref, tiT_ref, mk_ref, nf_ref, ef_ref,
          wen_ref, ben_ref, wee_ref, bee_ref,
          w12_ref, w3_ref, bm_ref,
          u1_ref, u2_ref, bu_ref,
          wf1_ref, bf1_ref, wf2_ref, bf2_ref,
          out_ref):
    i = pl.program_id(0)
    NB = _G * _PN
    EB = _G * _PE
    bf = jnp.bfloat16
    w12 = w12_ref[...].astype(bf)
    w3 = w3_ref[...].astype(bf)
    u1 = u1_ref[...].astype(bf)
    u2 = u2_ref[...].astype(bf)
    wf1 = wf1_ref[...].astype(bf)
    wf2 = wf2_ref[...].astype(bf)
    blocks = []
    for half in range(_H):
        es = slice(half * EB, (half + 1) * EB)
        ns = slice(half * NB, (half + 1) * NB)
        blocks += _group(
            fi_ref[0][es], ti_ref[0][es], tiT_ref[0][:, es], mk_ref[0][es],
            nf_ref[0][ns], ef_ref[0][es], i * _H * NB + half * NB,
            wen_ref[...], ben_ref[...], wee_ref[...], bee_ref[...],
            w12, w3, bm_ref[...], u1, u2, bu_ref[...],
            wf1, bf1_ref[...], wf2, bf2_ref[...])
    out_ref[...] = jnp.concatenate(blocks, axis=0)


_G2 = 128


def _sink_body(s_ref, out_ref):
    sin = s_ref[...]
    la = jnp.stack([jnp.transpose(sin[:, q, :]) for q in range(_CS)],
                   axis=0) / _TEMP

    for _ in range(_ITERS):
        m = jnp.max(la, axis=1, keepdims=True)
        la = la - (jnp.log(jnp.sum(jnp.exp(la - m), axis=1, keepdims=True)) + m)
        m = jnp.max(la, axis=0, keepdims=True)
        la = la - (jnp.log(jnp.sum(jnp.exp(la - m), axis=0, keepdims=True)) + m)

    p = jnp.exp(la)
    for q in range(_CS):
        out_ref[:, q, :] = jnp.transpose(p[q])


def kernel(node_features, edge_features, mask_from_idx,
           W_enc_n, b_enc_n, W_enc_e, b_enc_e,
           W_msg, b_msg, W_upd, b_upd,
           W_fc1, b_fc1, W_fc2, b_fc2,
           from_idx, to_idx, graph_idx):
    del graph_idx
    S = _B // (_G * _H)
    EB = _H * _G * _PE
    NB = _H * _G * _PN
    E_real = _B * _PE

    fi = from_idx[:E_real].astype(jnp.int32).reshape(S, EB, 1)
    ti3 = to_idx[:E_real].astype(jnp.int32).reshape(S, EB, 1)
    tiT = to_idx[:E_real].astype(jnp.int32).reshape(S, 1, EB)
    mk = mask_from_idx[:E_real].reshape(S, EB, 1)
    nf = node_features.reshape(S, NB, 1)
    ef = edge_features[:E_real].reshape(S, EB, 1)

    W12, W3 = W_msg[:2 * _D], W_msg[2 * _D:]
    U1, U2 = W_upd[:_D], W_upd[_D:]
    row = lambda v: v.reshape(1, -1)

    def step_spec(rows_, cols_):
        return pl.BlockSpec((1, rows_, cols_), lambda i: (i, 0, 0))

    def full_spec(shape):
        return pl.BlockSpec(shape, lambda i: tuple(0 for _ in shape))

    operands = [fi, ti3, tiT, mk, nf, ef,
                W_enc_n, row(b_enc_n), W_enc_e, row(b_enc_e),
                W12, W3, row(b_msg),
                U1, U2, row(b_upd),
                W_fc1, row(b_fc1), W_fc2, row(b_fc2)]
    specs = [step_spec(EB, 1), step_spec(EB, 1), step_spec(1, EB),
             step_spec(EB, 1), step_spec(NB, 1), step_spec(EB, 1)]
    specs += [full_spec(o.shape) for o in operands[6:]]

    s = pl.pallas_call(
        _body,
        grid=(S,),
        in_specs=specs,
        out_specs=pl.BlockSpec((_H * _G, _CS, _CS), lambda i: (i, 0, 0)),
        out_shape=jax.ShapeDtypeStruct((_B, _CS, _CS), jnp.float32),
        compiler_params=pltpu.CompilerParams(
            dimension_semantics=("parallel",)),
    )(*operands)

    return pl.pallas_call(
        _sink_body,
        grid=(_B // _G2,),
        in_specs=[pl.BlockSpec((_G2, _CS, _CS), lambda i: (i, 0, 0))],
        out_specs=pl.BlockSpec((_G2, _CS, _CS), lambda i: (i, 0, 0)),
        out_shape=jax.ShapeDtypeStruct((_B, _CS, _CS), jnp.float32),
        compiler_params=pltpu.CompilerParams(
            dimension_semantics=("parallel",)),
    )(s)
